# baseline (speedup 1.0000x reference)
import jax
import jax.numpy as jnp
from jax.experimental import pallas as pl
from jax.experimental.pallas import tpu as pltpu

_RB = 1024
_CB = 512
_NT = 8
_LOG2E = 1.4426950408889634


def _body(a_ref, bt_ref, u_ref, ut_ref, o_ref, ot_ref):
    nlast = _NT // 2 - 1
    s = pl.program_id(0)

    @pl.when(s == 0)
    def _init():
        o_ref[...] = jnp.zeros_like(o_ref)
        ot_ref[...] = jnp.zeros_like(ot_ref)

    for t in range(_NT + 1):
        first = t < _NT - s
        i_t = jnp.where(first, s, _NT - 1 - s)
        j_t = jnp.where(first, s + t, t - 1)
        row = i_t * _RB
        a = a_ref[pl.ds(row, _RB), :]
        a16 = a.astype(jnp.bfloat16)
        ci = jnp.sum(a * a, axis=1, keepdims=True) * (0.5 * _LOG2E)
        mirror = jnp.where(j_t > i_t, 1.0, 0.0)
        for k in range(_RB // _CB):
            col = j_t * _RB + k * _CB
            bt = bt_ref[:, pl.ds(col, _CB)]
            cj = jnp.sum(bt * bt, axis=0, keepdims=True) * (0.5 * _LOG2E)
            mm = jax.lax.dot_general(
                a16, bt.astype(jnp.bfloat16),
                (((1,), (0,)), ((), ())),
                preferred_element_type=jnp.float32)
            w = jnp.exp2(jnp.minimum(mm * _LOG2E - (ci + cj), 0.0)
                         ).astype(jnp.bfloat16)
            o_ref[pl.ds(row, _RB), :] += jax.lax.dot_general(
                w, u_ref[pl.ds(col, _CB), :],
                (((1,), (0,)), ((), ())),
                preferred_element_type=jnp.float32)
            ot_ref[:, pl.ds(col, _CB)] += mirror * jax.lax.dot_general(
                ut_ref[:, pl.ds(row, _RB)], w,
                (((1,), (0,)), ((), ())),
                preferred_element_type=jnp.float32)

    @pl.when(s == nlast)
    def _finish():
        o_ref[...] += ot_ref[...].T - u_ref[...]


def kernel(U, ref):
    n, c = U.shape
    refp = jnp.pad(ref, ((0, 0), (0, 8 - ref.shape[1])))
    refT = refp.T
    UT = U.T

    out = pl.pallas_call(
        _body,
        grid=(_NT // 2,),
        in_specs=[
            pl.BlockSpec((n, 8), lambda i: (0, 0)),
            pl.BlockSpec((8, n), lambda i: (0, 0)),
            pl.BlockSpec((n, c), lambda i: (0, 0)),
            pl.BlockSpec((c, n), lambda i: (0, 0)),
        ],
        out_specs=pl.BlockSpec((n, c), lambda i: (0, 0)),
        out_shape=jax.ShapeDtypeStruct((n, c), jnp.float32),
        scratch_shapes=[pltpu.VMEM((c, n), jnp.float32)],
    )(refp, refT, U, UT)
    return out


# speedup vs baseline: 1.0203x; 1.0203x over previous
import jax
import jax.numpy as jnp
from jax.experimental import pallas as pl

_RB = 1024
_CB = 512
_NT = 8
_LOG2E = 1.4426950408889634


def _body(a_ref, bt_ref, u_ref, o_ref):
    s = pl.program_id(0)

    @pl.when(s == 0)
    def _init():
        o_ref[...] = jnp.zeros_like(o_ref)

    for t in range(_NT + 1):
        first = t < _NT - s
        i_t = jnp.where(first, s, _NT - 1 - s)
        j_t = jnp.where(first, s + t, t - 1)
        row = i_t * _RB
        a = a_ref[pl.ds(row, _RB), :]
        a16 = a.astype(jnp.bfloat16)
        ci = jnp.sum(a * a, axis=1, keepdims=True) * (0.5 * _LOG2E)
        mirror = jnp.where(j_t > i_t, 1.0, 0.0)
        for k in range(_RB // _CB):
            col = j_t * _RB + k * _CB
            bt = bt_ref[:, pl.ds(col, _CB)]
            cj = jnp.sum(bt * bt, axis=0, keepdims=True) * (0.5 * _LOG2E)
            mm = jax.lax.dot_general(
                a16, bt.astype(jnp.bfloat16),
                (((1,), (0,)), ((), ())),
                preferred_element_type=jnp.float32)
            w = jnp.exp2(jnp.minimum(mm * _LOG2E - (ci + cj), 0.0)
                         ).astype(jnp.bfloat16)
            o_ref[pl.ds(row, _RB), :] += jax.lax.dot_general(
                w, u_ref[pl.ds(col, _CB), :],
                (((1,), (0,)), ((), ())),
                preferred_element_type=jnp.float32)
            o_ref[pl.ds(col, _CB), :] += mirror * jax.lax.dot_general(
                w, u_ref[pl.ds(row, _RB), :],
                (((0,), (0,)), ((), ())),
                preferred_element_type=jnp.float32)

    @pl.when(s == _NT // 2 - 1)
    def _finish():
        o_ref[...] -= u_ref[...]


def kernel(U, ref):
    n, c = U.shape
    refp = jnp.pad(ref, ((0, 0), (0, 8 - ref.shape[1])))
    refT = refp.T

    out = pl.pallas_call(
        _body,
        grid=(_NT // 2,),
        in_specs=[
            pl.BlockSpec((n, 8), lambda i: (0, 0)),
            pl.BlockSpec((8, n), lambda i: (0, 0)),
            pl.BlockSpec((n, c), lambda i: (0, 0)),
        ],
        out_specs=pl.BlockSpec((n, c), lambda i: (0, 0)),
        out_shape=jax.ShapeDtypeStruct((n, c), jnp.float32),
    )(refp, refT, U)
    return out


# speedup vs baseline: 1.0344x; 1.0138x over previous
import jax
import jax.numpy as jnp
from jax.experimental import pallas as pl

_RB = 1024
_CB = 512
_NT = 8
_LOG2E = 1.4426950408889634


def _body(a_ref, c_ref, u_ref, o_ref):
    s = pl.program_id(0)

    @pl.when(s == 0)
    def _init():
        o_ref[...] = jnp.zeros_like(o_ref)

    for t in range(_NT + 1):
        first = t < _NT - s
        i_t = jnp.where(first, s, _NT - 1 - s)
        j_t = jnp.where(first, s + t, t - 1)
        row = i_t * _RB
        a = a_ref[pl.ds(row, _RB), :]
        a16 = a.astype(jnp.bfloat16)
        ci = jnp.sum(a * a, axis=1, keepdims=True) * (0.5 * _LOG2E)
        mirror = jnp.where(j_t > i_t, 1.0, 0.0)
        for k in range(_RB // _CB):
            col = j_t * _RB + k * _CB
            b16 = a_ref[pl.ds(col, _CB), :].astype(jnp.bfloat16)
            cj = c_ref[:, pl.ds(col, _CB)]
            mm = jax.lax.dot_general(
                a16, b16, (((1,), (1,)), ((), ())),
                preferred_element_type=jnp.float32)
            w = jnp.exp2(jnp.minimum(mm * _LOG2E - (ci + cj), 0.0)
                         ).astype(jnp.bfloat16)
            o_ref[pl.ds(row, _RB), :] += jax.lax.dot_general(
                w, u_ref[pl.ds(col, _CB), :],
                (((1,), (0,)), ((), ())),
                preferred_element_type=jnp.float32)
            o_ref[pl.ds(col, _CB), :] += mirror * jax.lax.dot_general(
                w, u_ref[pl.ds(row, _RB), :],
                (((0,), (0,)), ((), ())),
                preferred_element_type=jnp.float32)

    @pl.when(s == _NT // 2 - 1)
    def _finish():
        o_ref[...] -= u_ref[...]


def kernel(U, ref):
    n, c = U.shape
    crow = (jnp.sum(ref * ref, axis=1) * (0.5 * _LOG2E)).reshape(1, n)

    out = pl.pallas_call(
        _body,
        grid=(_NT // 2,),
        in_specs=[
            pl.BlockSpec((n, ref.shape[1]), lambda i: (0, 0)),
            pl.BlockSpec((1, n), lambda i: (0, 0)),
            pl.BlockSpec((n, c), lambda i: (0, 0)),
        ],
        out_specs=pl.BlockSpec((n, c), lambda i: (0, 0)),
        out_shape=jax.ShapeDtypeStruct((n, c), jnp.float32),
    )(ref, crow, U)
    return out


# speedup vs baseline: 1.0450x; 1.0103x over previous
import jax
import jax.numpy as jnp
from jax.experimental import pallas as pl

_RB = 1024
_CB = 512
_NT = 8
_LOG2E = 1.4426950408889634


def _tiles_for(s):
    rows = [s, _NT - 1 - s, 2 + s, _NT - 3 - s]
    return [(i, j) for i in rows for j in range(i, _NT)]


_T0 = _tiles_for(0)
_T1 = _tiles_for(1)


def _body(a_ref, c_ref, u_ref, o_ref):
    s = pl.program_id(0)

    @pl.when(s == 0)
    def _init():
        o_ref[...] = jnp.zeros_like(o_ref)

    for t in range(len(_T0)):
        i_t = jnp.where(s == 0, _T0[t][0], _T1[t][0])
        j_t = jnp.where(s == 0, _T0[t][1], _T1[t][1])
        row = i_t * _RB
        a = a_ref[pl.ds(row, _RB), :]
        a16 = a.astype(jnp.bfloat16)
        ci = jnp.sum(a * a, axis=1, keepdims=True) * (0.5 * _LOG2E)
        mirror = jnp.where(j_t > i_t, 1.0, 0.0)
        for k in range(_RB // _CB):
            col = j_t * _RB + k * _CB
            b16 = a_ref[pl.ds(col, _CB), :].astype(jnp.bfloat16)
            cj = c_ref[:, pl.ds(col, _CB)]
            mm = jax.lax.dot_general(
                a16, b16, (((1,), (1,)), ((), ())),
                preferred_element_type=jnp.float32)
            w = jnp.exp2(jnp.minimum(mm * _LOG2E - (ci + cj), 0.0)
                         ).astype(jnp.bfloat16)
            o_ref[pl.ds(row, _RB), :] += jax.lax.dot_general(
                w, u_ref[pl.ds(col, _CB), :],
                (((1,), (0,)), ((), ())),
                preferred_element_type=jnp.float32)
            o_ref[pl.ds(col, _CB), :] += mirror * jax.lax.dot_general(
                w, u_ref[pl.ds(row, _RB), :],
                (((0,), (0,)), ((), ())),
                preferred_element_type=jnp.float32)

    @pl.when(s == 1)
    def _finish():
        o_ref[...] -= u_ref[...]


def kernel(U, ref):
    n, c = U.shape
    crow = (jnp.sum(ref * ref, axis=1) * (0.5 * _LOG2E)).reshape(1, n)

    out = pl.pallas_call(
        _body,
        grid=(2,),
        in_specs=[
            pl.BlockSpec((n, ref.shape[1]), lambda i: (0, 0)),
            pl.BlockSpec((1, n), lambda i: (0, 0)),
            pl.BlockSpec((n, c), lambda i: (0, 0)),
        ],
        out_specs=pl.BlockSpec((n, c), lambda i: (0, 0)),
        out_shape=jax.ShapeDtypeStruct((n, c), jnp.float32),
    )(ref, crow, U)
    return out
